# baseline (speedup 1.0000x reference)
import functools

import jax
import jax.numpy as jnp
from jax import lax
from jax.experimental import pallas as pl
from jax.experimental.pallas import tpu as pltpu
from jax.experimental.pallas import tpu_sc as plsc

N = 10000
E = 320000
D_IN = 128
HEADS = 8
DIM = 8
D_OUT = 128

NP = 10240
NBLK = NP // 256
NC = 2
NS = 16
RPT = NP // NS
ET = E + N
K = 96
TE = ((ET // (NC * NS) + K) // K) * K
EP = TE * NC * NS
CH = TE // K

_mesh = plsc.VectorSubcoreMesh(
    core_axis_name="c", subcore_axis_name="s", num_cores=NC, num_subcores=NS
)

_GDN = lax.GatherDimensionNumbers(
    offset_dims=(), collapsed_slice_dims=(0,), start_index_map=(0,)
)


def _vperm(v, idx16):
    return lax.gather(
        v, idx16.reshape(16, 1), _GDN, (1,),
        mode=lax.GatherScatterMode.PROMISE_IN_BOUNDS,
    )


def _tc_pre_body(x_ref, w1_ref, as_ref, ad_ref, h_out, asd_out):
    h = jnp.dot(x_ref[...], w1_ref[...], preferred_element_type=jnp.float32)
    a_s = jnp.dot(h, as_ref[...], preferred_element_type=jnp.float32)
    a_d = jnp.dot(h, ad_ref[...], preferred_element_type=jnp.float32)
    h_out[...] = h
    asd_out[...] = jnp.concatenate([a_s, a_d], axis=1)


def _tc_pre(xp, W1, As, Ad):
    return pl.pallas_call(
        _tc_pre_body,
        grid=(NBLK,),
        in_specs=[
            pl.BlockSpec((256, D_IN), lambda i: (i, 0)),
            pl.BlockSpec((D_IN, HEADS * DIM), lambda i: (0, 0)),
            pl.BlockSpec((HEADS * DIM, HEADS), lambda i: (0, 0)),
            pl.BlockSpec((HEADS * DIM, HEADS), lambda i: (0, 0)),
        ],
        out_specs=[
            pl.BlockSpec((256, HEADS * DIM), lambda i: (i, 0)),
            pl.BlockSpec((256, 16), lambda i: (i, 0)),
        ],
        out_shape=[
            jax.ShapeDtypeStruct((NP, HEADS * DIM), jnp.float32),
            jax.ShapeDtypeStruct((NP, 16), jnp.float32),
        ],
    )(xp, W1, As, Ad)


def _sc1_body(src_hbm, dst_hbm, asd_hbm, h1_hbm, acc_out,
              acc_sp, src_i, dst_i, gs, gd, gh, msgex, zbuf, sem):
    c = lax.axis_index("c")
    s = lax.axis_index("s")
    lane = lax.iota(jnp.int32, 16)
    lo8 = lane < 8
    shift8 = (lane & 7) + 8
    hi_half = lane >> 3
    zero16 = jnp.zeros((16,), jnp.float32)

    def _z(r, _):
        for j in range(5):
            zbuf[r, pl.ds(16 * j, 16)] = zero16
        return 0
    lax.fori_loop(0, 64, _z, 0)
    rows0 = s * RPT
    for i in range(RPT // 64):
        pltpu.sync_copy(zbuf, acc_sp.at[pl.ds(rows0 + 64 * i, 64)])
    plsc.subcore_barrier()

    ebase0 = (c * NS + s) * TE

    def _scat(q):
        pltpu.sync_copy(msgex.at[pl.ds(q * K, K)], acc_sp.at[dst_i.at[q]],
                        add=True)

    def _iter(i, _):
        for q in range(2):
            ch_i = 2 * i + q
            eb = ebase0 + ch_i * K
            pltpu.sync_copy(src_hbm.at[pl.ds(eb, K)], src_i.at[0])
            pltpu.sync_copy(dst_hbm.at[pl.ds(eb, K)], dst_i.at[q])
            d1 = pltpu.async_copy(asd_hbm.at[src_i.at[0]], gs, sem)
            d2 = pltpu.async_copy(asd_hbm.at[dst_i.at[q]], gd, sem)
            d3 = pltpu.async_copy(h1_hbm.at[src_i.at[0]], gh, sem)
            if q == 1:
                _scat(0)
            else:
                @pl.when(ch_i >= 1)
                def _():
                    _scat(1)
            d1.wait()
            d2.wait()
            d3.wait()

            @plsc.parallel_loop(0, K, unroll=8)
            def _edge(k):
                e16 = gs[k] + _vperm(gd[k], shift8)
                e16 = jnp.where(e16 >= 0, e16, 0.2 * e16)
                ex = jnp.where(lo8, jnp.exp(e16), 0.0)
                msgex[q * K + k, pl.ds(64, 16)] = ex
                for j in range(4):
                    w = _vperm(ex, hi_half + 2 * j)
                    msgex[q * K + k, pl.ds(16 * j, 16)] = (
                        w * gh[k, pl.ds(16 * j, 16)])
        return 0

    lax.fori_loop(0, CH // 2, _iter, 0)
    _scat(1)
    plsc.subcore_barrier()

    pltpu.sync_copy(acc_sp.at[pl.ds(rows0, RPT)],
                    acc_out.at[c, pl.ds(rows0, RPT)])


_sc1 = pl.kernel(
    _sc1_body,
    out_type=(
        jax.ShapeDtypeStruct((NC, NP, 80), jnp.float32),
    ),
    mesh=_mesh,
    scratch_types=(
        pltpu.VMEM_SHARED((NP, 80), jnp.float32),
        pltpu.VMEM((1, K), jnp.int32),
        pltpu.VMEM((2, K), jnp.int32),
        pltpu.VMEM((K, 16), jnp.float32),
        pltpu.VMEM((K, 16), jnp.float32),
        pltpu.VMEM((K, 64), jnp.float32),
        pltpu.VMEM((2 * K, 80), jnp.float32),
        pltpu.VMEM((64, 80), jnp.float32),
        pltpu.SemaphoreType.DMA,
    ),
    compiler_params=pltpu.CompilerParams(use_tc_tiling_on_sc=False),
)


def _tc_mid_body(acc_ref, b1_ref, w2_ref, ps_ref, pd_ref, rep_ref,
                 r_out, asd_out):
    acc = acc_ref[0] + acc_ref[1]
    agg = acc[:, 0:64]
    den = acc[:, 64:80]
    dexp = jnp.dot(den, rep_ref[...], preferred_element_type=jnp.float32)
    r = agg / (dexp + 1e-16) + b1_ref[...]
    r = jnp.maximum(r, 0.0)
    vs2 = jnp.dot(w2_ref[...], ps_ref[...], preferred_element_type=jnp.float32)
    vd2 = jnp.dot(w2_ref[...], pd_ref[...], preferred_element_type=jnp.float32)
    a_s = jnp.dot(r, vs2, preferred_element_type=jnp.float32)
    a_d = jnp.dot(r, vd2, preferred_element_type=jnp.float32)
    r_out[...] = r
    asd_out[...] = jnp.concatenate([a_s, a_d], axis=1)


def _tc_mid(acc1, b1r, W2, Ps, Pd, Rep):
    return pl.pallas_call(
        _tc_mid_body,
        grid=(NBLK,),
        in_specs=[
            pl.BlockSpec((NC, 256, 80), lambda i: (0, i, 0)),
            pl.BlockSpec((1, 64), lambda i: (0, 0)),
            pl.BlockSpec((64, HEADS * D_OUT), lambda i: (0, 0)),
            pl.BlockSpec((HEADS * D_OUT, HEADS), lambda i: (0, 0)),
            pl.BlockSpec((HEADS * D_OUT, HEADS), lambda i: (0, 0)),
            pl.BlockSpec((16, 64), lambda i: (0, 0)),
        ],
        out_specs=[
            pl.BlockSpec((256, 64), lambda i: (i, 0)),
            pl.BlockSpec((256, 16), lambda i: (i, 0)),
        ],
        out_shape=[
            jax.ShapeDtypeStruct((NP, 64), jnp.float32),
            jax.ShapeDtypeStruct((NP, 16), jnp.float32),
        ],
    )(acc1, b1r, W2, Ps, Pd, Rep)


def _sc2_body(src_hbm, dst_hbm, asd_hbm, rb_hbm, agg_out, den_out, ex_out,
              agg_sp, den_sp, src_i, dst_i, rsrc_i, gs, gd, gr, exb, msg,
              zbuf, zb16, sem):
    c = lax.axis_index("c")
    s = lax.axis_index("s")
    lane = lax.iota(jnp.int32, 16)
    lo8 = lane < 8
    shift8 = (lane & 7) + 8
    lane0 = lane & 0
    zero16 = jnp.zeros((16,), jnp.float32)

    def _z(r, _):
        for j in range(8):
            zbuf[r, pl.ds(16 * j, 16)] = zero16
        zb16[r] = zero16
        return 0
    lax.fori_loop(0, 16, _z, 0)
    rows0 = s * RPT
    ebase0 = (c * NS + s) * TE

    for p in range(4):
        for i in range(RPT // 16):
            pltpu.sync_copy(zbuf, agg_sp.at[pl.ds(rows0 + 16 * i, 16)])
            if p == 0:
                pltpu.sync_copy(zb16, den_sp.at[pl.ds(rows0 + 16 * i, 16)])
        plsc.subcore_barrier()

        def _scat(q, eb_prev):
            pltpu.sync_copy(msg.at[pl.ds(q * K, K)], agg_sp.at[dst_i.at[q]],
                            add=True)

        def _iter(i, _):
            for q in range(2):
                ch_i = 2 * i + q
                eb = ebase0 + ch_i * K
                pltpu.sync_copy(src_hbm.at[pl.ds(eb, K)], src_i.at[0])
                pltpu.sync_copy(dst_hbm.at[pl.ds(eb, K)], dst_i.at[q])
                for u in range(K // 16):
                    v = src_i[0, pl.ds(16 * u, 16)]
                    rsrc_i[0, pl.ds(16 * u, 16)] = v * 4 + p
                dd = pl.ds(q * K, K)
                if p == 0:
                    d1 = pltpu.async_copy(asd_hbm.at[src_i.at[0]], gs, sem)
                    d2 = pltpu.async_copy(asd_hbm.at[dst_i.at[q]], gd, sem)
                else:
                    d1 = pltpu.async_copy(ex_out.at[pl.ds(eb, K)], exb, sem)
                    d2 = None
                d3 = pltpu.async_copy(rb_hbm.at[rsrc_i.at[0]], gr, sem)
                if q == 1:
                    _scat(0, eb - K)
                else:
                    @pl.when(ch_i >= 1)
                    def _():
                        _scat(1, eb - K)
                d1.wait()
                if d2 is not None:
                    d2.wait()
                d3.wait()

                @plsc.parallel_loop(0, K, unroll=8)
                def _edge(k):
                    rrow = gr[k]
                    if p == 0:
                        e16 = gs[k] + _vperm(gd[k], shift8)
                        e16 = jnp.where(e16 >= 0, e16, 0.2 * e16)
                        ex = jnp.where(lo8, jnp.exp(e16), 0.0)
                        exb[k] = ex
                        for h in range(8):
                            w = _vperm(ex, lane0 + h)
                            msg[q * K + k, pl.ds(16 * h, 16)] = w * rrow
                    else:
                        ex = exb[k]
                        for h in range(8):
                            msg[q * K + k, pl.ds(16 * h, 16)] = ex[h] * rrow

                if p == 0:
                    pltpu.sync_copy(exb, den_sp.at[dst_i.at[q]], add=True)
                    pltpu.sync_copy(exb, ex_out.at[pl.ds(eb, K)])
            return 0

        lax.fori_loop(0, CH // 2, _iter, 0)
        _scat(1, ebase0 + (CH - 1) * K)
        plsc.subcore_barrier()

        pltpu.sync_copy(agg_sp.at[pl.ds(rows0, RPT)],
                        agg_out.at[c, p, pl.ds(rows0, RPT)])
        if p == 0:
            pltpu.sync_copy(den_sp.at[pl.ds(rows0, RPT)],
                            den_out.at[c, pl.ds(rows0, RPT)])
        plsc.subcore_barrier()


_sc2 = pl.kernel(
    _sc2_body,
    out_type=(
        jax.ShapeDtypeStruct((NC, 4, NP, 128), jnp.float32),
        jax.ShapeDtypeStruct((NC, NP, 16), jnp.float32),
        jax.ShapeDtypeStruct((EP, 16), jnp.float32),
    ),
    mesh=_mesh,
    scratch_types=(
        pltpu.VMEM_SHARED((NP, 128), jnp.float32),
        pltpu.VMEM_SHARED((NP, 16), jnp.float32),
        pltpu.VMEM((1, K), jnp.int32),
        pltpu.VMEM((2, K), jnp.int32),
        pltpu.VMEM((1, K), jnp.int32),
        pltpu.VMEM((K, 16), jnp.float32),
        pltpu.VMEM((K, 16), jnp.float32),
        pltpu.VMEM((K, 16), jnp.float32),
        pltpu.VMEM((K, 16), jnp.float32),
        pltpu.VMEM((2 * K, 128), jnp.float32),
        pltpu.VMEM((16, 128), jnp.float32),
        pltpu.VMEM((16, 16), jnp.float32),
        pltpu.SemaphoreType.DMA,
    ),
    compiler_params=pltpu.CompilerParams(use_tc_tiling_on_sc=False),
)


def _tc_fin_body(agg_ref, den_ref, w2_ref, b2_ref, out_ref):
    agg = agg_ref[0] + agg_ref[1]
    den = den_ref[0] + den_ref[1]
    inv = 0.125 / (den + 1e-16)
    acc = jnp.zeros((256, D_OUT), jnp.float32) + b2_ref[...]
    for h in range(HEADS):
        ah = jnp.concatenate(
            [agg[p, :, 16 * h:16 * h + 16] for p in range(4)], axis=1
        )
        ah = ah * inv[:, h:h + 1]
        acc = acc + jnp.dot(ah, w2_ref[:, 128 * h:128 * h + 128],
                            preferred_element_type=jnp.float32)
    out_ref[...] = acc


def _tc_fin(agg2, den2, W2, b2r):
    return pl.pallas_call(
        _tc_fin_body,
        grid=(NBLK,),
        in_specs=[
            pl.BlockSpec((NC, 4, 256, 128), lambda i: (0, 0, i, 0)),
            pl.BlockSpec((NC, 256, 16), lambda i: (0, i, 0)),
            pl.BlockSpec((64, HEADS * D_OUT), lambda i: (0, 0)),
            pl.BlockSpec((1, D_OUT), lambda i: (0, 0)),
        ],
        out_specs=pl.BlockSpec((256, D_OUT), lambda i: (i, 0)),
        out_shape=jax.ShapeDtypeStruct((NP, D_OUT), jnp.float32),
    )(agg2, den2, W2, b2r)


def kernel(x, edge_index, W1, att_src1, att_dst1, b1,
           W2, att_src2, att_dst2, b2):
    f32 = jnp.float32
    x = x.astype(f32)

    loop = jnp.arange(N, dtype=jnp.int32)
    src = jnp.concatenate([edge_index[0].astype(jnp.int32), loop])
    dst = jnp.concatenate([edge_index[1].astype(jnp.int32), loop])
    pad = EP - ET
    srcp = jnp.concatenate([src, jnp.zeros((pad,), jnp.int32)])
    dstp = jnp.concatenate([dst, jnp.full((pad,), NP - 1, jnp.int32)])

    hd = HEADS * DIM
    As = jnp.zeros((hd, HEADS), f32)
    Ad = jnp.zeros((hd, HEADS), f32)
    hh = jnp.arange(hd) // DIM
    As = As.at[jnp.arange(hd), hh].set(att_src1.reshape(-1).astype(f32))
    Ad = Ad.at[jnp.arange(hd), hh].set(att_dst1.reshape(-1).astype(f32))
    ho = HEADS * D_OUT
    hh2 = jnp.arange(ho) // D_OUT
    Ps = jnp.zeros((ho, HEADS), f32)
    Pd = jnp.zeros((ho, HEADS), f32)
    Ps = Ps.at[jnp.arange(ho), hh2].set(att_src2.reshape(-1).astype(f32))
    Pd = Pd.at[jnp.arange(ho), hh2].set(att_dst2.reshape(-1).astype(f32))
    Rep = jnp.zeros((16, 64), f32)
    Rep = Rep.at[jnp.arange(64) // DIM, jnp.arange(64)].set(1.0)

    xp = jnp.pad(x, ((0, NP - N), (0, 0)))
    h1p, asd1 = _tc_pre(xp, W1.astype(f32), As, Ad)

    (acc1,) = _sc1(srcp, dstp, asd1, h1p)

    rfull, asd2 = _tc_mid(acc1, b1.astype(f32).reshape(1, 64),
                          W2.astype(f32), Ps, Pd, Rep)
    rb = rfull.reshape(NP * 4, 16)

    agg2, den2, _ = _sc2(srcp, dstp, asd2, rb)

    outp = _tc_fin(agg2, den2, W2.astype(f32),
                   b2.astype(f32).reshape(1, D_OUT))
    return outp[:N]


# speedup vs baseline: 1.0321x; 1.0321x over previous
import functools

import jax
import jax.numpy as jnp
from jax import lax
from jax.experimental import pallas as pl
from jax.experimental.pallas import tpu as pltpu
from jax.experimental.pallas import tpu_sc as plsc

N = 10000
E = 320000
D_IN = 128
HEADS = 8
DIM = 8
D_OUT = 128

NP = 10240
NBLK = NP // 256
NC = 2
NS = 16
RPT = NP // NS
ET = E + N
K = 96
TE = ((ET // (NC * NS) + K) // K) * K
EP = TE * NC * NS
CH = TE // K

_mesh = plsc.VectorSubcoreMesh(
    core_axis_name="c", subcore_axis_name="s", num_cores=NC, num_subcores=NS
)

_GDN = lax.GatherDimensionNumbers(
    offset_dims=(), collapsed_slice_dims=(0,), start_index_map=(0,)
)


def _vperm(v, idx16):
    return lax.gather(
        v, idx16.reshape(16, 1), _GDN, (1,),
        mode=lax.GatherScatterMode.PROMISE_IN_BOUNDS,
    )


def _tc_pre_body(x_ref, w1_ref, as_ref, ad_ref, h_out, asd_out):
    h = jnp.dot(x_ref[...], w1_ref[...], preferred_element_type=jnp.float32)
    a_s = jnp.dot(h, as_ref[...], preferred_element_type=jnp.float32)
    a_d = jnp.dot(h, ad_ref[...], preferred_element_type=jnp.float32)
    h_out[...] = h
    asd_out[...] = jnp.concatenate([a_s, a_d], axis=1)


def _tc_pre(xp, W1, As, Ad):
    return pl.pallas_call(
        _tc_pre_body,
        grid=(NBLK,),
        in_specs=[
            pl.BlockSpec((256, D_IN), lambda i: (i, 0)),
            pl.BlockSpec((D_IN, HEADS * DIM), lambda i: (0, 0)),
            pl.BlockSpec((HEADS * DIM, HEADS), lambda i: (0, 0)),
            pl.BlockSpec((HEADS * DIM, HEADS), lambda i: (0, 0)),
        ],
        out_specs=[
            pl.BlockSpec((256, HEADS * DIM), lambda i: (i, 0)),
            pl.BlockSpec((256, 16), lambda i: (i, 0)),
        ],
        out_shape=[
            jax.ShapeDtypeStruct((NP, HEADS * DIM), jnp.float32),
            jax.ShapeDtypeStruct((NP, 16), jnp.float32),
        ],
    )(xp, W1, As, Ad)


def _sc1_body(src_hbm, dst_hbm, asd_hbm, h1_hbm, acc_out,
              acc_sp, src_i, dst_i, gs, gd, gh, msgex, zbuf, sem):
    c = lax.axis_index("c")
    s = lax.axis_index("s")
    lane = lax.iota(jnp.int32, 16)
    lo8 = lane < 8
    shift8 = (lane & 7) + 8
    hi_half = lane >> 3
    zero16 = jnp.zeros((16,), jnp.float32)

    def _z(r, _):
        for j in range(5):
            zbuf[r, pl.ds(16 * j, 16)] = zero16
        return 0
    lax.fori_loop(0, 64, _z, 0)
    rows0 = s * RPT
    for i in range(RPT // 64):
        pltpu.sync_copy(zbuf, acc_sp.at[pl.ds(rows0 + 64 * i, 64)])
    plsc.subcore_barrier()

    ebase0 = (c * NS + s) * TE

    def _scat(q):
        pltpu.sync_copy(msgex.at[pl.ds(q * K, K)], acc_sp.at[dst_i.at[q]],
                        add=True)

    def _iter(i, _):
        for q in range(2):
            ch_i = 2 * i + q
            eb = ebase0 + ch_i * K
            pltpu.sync_copy(src_hbm.at[pl.ds(eb, K)], src_i.at[0])
            pltpu.sync_copy(dst_hbm.at[pl.ds(eb, K)], dst_i.at[q])
            d1 = pltpu.async_copy(asd_hbm.at[src_i.at[0]], gs, sem)
            d2 = pltpu.async_copy(asd_hbm.at[dst_i.at[q]], gd, sem)
            d3 = pltpu.async_copy(h1_hbm.at[src_i.at[0]], gh, sem)
            if q == 1:
                _scat(0)
            else:
                @pl.when(ch_i >= 1)
                def _():
                    _scat(1)
            d1.wait()
            d2.wait()
            d3.wait()

            @plsc.parallel_loop(0, K, unroll=4)
            def _edge(k):
                e16 = gs[k] + _vperm(gd[k], shift8)
                e16 = jnp.where(e16 >= 0, e16, 0.2 * e16)
                ex = jnp.where(lo8, jnp.exp(e16), 0.0)
                msgex[q * K + k, pl.ds(64, 16)] = ex
                for j in range(4):
                    w = _vperm(ex, hi_half + 2 * j)
                    msgex[q * K + k, pl.ds(16 * j, 16)] = (
                        w * gh[k, pl.ds(16 * j, 16)])
        return 0

    lax.fori_loop(0, CH // 2, _iter, 0)
    _scat(1)
    plsc.subcore_barrier()

    pltpu.sync_copy(acc_sp.at[pl.ds(rows0, RPT)],
                    acc_out.at[c, pl.ds(rows0, RPT)])


_sc1 = pl.kernel(
    _sc1_body,
    out_type=(
        jax.ShapeDtypeStruct((NC, NP, 80), jnp.float32),
    ),
    mesh=_mesh,
    scratch_types=(
        pltpu.VMEM_SHARED((NP, 80), jnp.float32),
        pltpu.VMEM((1, K), jnp.int32),
        pltpu.VMEM((2, K), jnp.int32),
        pltpu.VMEM((K, 16), jnp.float32),
        pltpu.VMEM((K, 16), jnp.float32),
        pltpu.VMEM((K, 64), jnp.float32),
        pltpu.VMEM((2 * K, 80), jnp.float32),
        pltpu.VMEM((64, 80), jnp.float32),
        pltpu.SemaphoreType.DMA,
    ),
    compiler_params=pltpu.CompilerParams(use_tc_tiling_on_sc=False),
)


def _tc_mid_body(acc_ref, b1_ref, w2_ref, ps_ref, pd_ref, rep_ref,
                 r_out, asd_out):
    acc = acc_ref[0] + acc_ref[1]
    agg = acc[:, 0:64]
    den = acc[:, 64:80]
    dexp = jnp.dot(den, rep_ref[...], preferred_element_type=jnp.float32)
    r = agg / (dexp + 1e-16) + b1_ref[...]
    r = jnp.maximum(r, 0.0)
    vs2 = jnp.dot(w2_ref[...], ps_ref[...], preferred_element_type=jnp.float32)
    vd2 = jnp.dot(w2_ref[...], pd_ref[...], preferred_element_type=jnp.float32)
    a_s = jnp.dot(r, vs2, preferred_element_type=jnp.float32)
    a_d = jnp.dot(r, vd2, preferred_element_type=jnp.float32)
    r_out[...] = r
    asd_out[...] = jnp.concatenate([a_s, a_d], axis=1)


def _tc_mid(acc1, b1r, W2, Ps, Pd, Rep):
    return pl.pallas_call(
        _tc_mid_body,
        grid=(NBLK,),
        in_specs=[
            pl.BlockSpec((NC, 256, 80), lambda i: (0, i, 0)),
            pl.BlockSpec((1, 64), lambda i: (0, 0)),
            pl.BlockSpec((64, HEADS * D_OUT), lambda i: (0, 0)),
            pl.BlockSpec((HEADS * D_OUT, HEADS), lambda i: (0, 0)),
            pl.BlockSpec((HEADS * D_OUT, HEADS), lambda i: (0, 0)),
            pl.BlockSpec((16, 64), lambda i: (0, 0)),
        ],
        out_specs=[
            pl.BlockSpec((256, 64), lambda i: (i, 0)),
            pl.BlockSpec((256, 16), lambda i: (i, 0)),
        ],
        out_shape=[
            jax.ShapeDtypeStruct((NP, 64), jnp.float32),
            jax.ShapeDtypeStruct((NP, 16), jnp.float32),
        ],
    )(acc1, b1r, W2, Ps, Pd, Rep)


def _sc2_body(src_hbm, dst_hbm, asd_hbm, rb_hbm, agg_out, den_out, ex_out,
              agg_sp, den_sp, src_i, dst_i, rsrc_i, gs, gd, gr, exb, msg,
              zbuf, zb16, sem):
    c = lax.axis_index("c")
    s = lax.axis_index("s")
    lane = lax.iota(jnp.int32, 16)
    lo8 = lane < 8
    shift8 = (lane & 7) + 8
    lane0 = lane & 0
    zero16 = jnp.zeros((16,), jnp.float32)

    def _z(r, _):
        for j in range(8):
            zbuf[r, pl.ds(16 * j, 16)] = zero16
        zb16[r] = zero16
        return 0
    lax.fori_loop(0, 16, _z, 0)
    rows0 = s * RPT
    ebase0 = (c * NS + s) * TE

    for p in range(4):
        for i in range(RPT // 16):
            pltpu.sync_copy(zbuf, agg_sp.at[pl.ds(rows0 + 16 * i, 16)])
            if p == 0:
                pltpu.sync_copy(zb16, den_sp.at[pl.ds(rows0 + 16 * i, 16)])
        plsc.subcore_barrier()

        def _scat(q, eb_prev):
            pltpu.sync_copy(msg.at[pl.ds(q * K, K)], agg_sp.at[dst_i.at[q]],
                            add=True)

        def _iter(i, _):
            for q in range(2):
                ch_i = 2 * i + q
                eb = ebase0 + ch_i * K
                pltpu.sync_copy(src_hbm.at[pl.ds(eb, K)], src_i.at[0])
                pltpu.sync_copy(dst_hbm.at[pl.ds(eb, K)], dst_i.at[q])
                for u in range(K // 16):
                    v = src_i[0, pl.ds(16 * u, 16)]
                    rsrc_i[0, pl.ds(16 * u, 16)] = v * 4 + p
                dd = pl.ds(q * K, K)
                if p == 0:
                    d1 = pltpu.async_copy(asd_hbm.at[src_i.at[0]], gs, sem)
                    d2 = pltpu.async_copy(asd_hbm.at[dst_i.at[q]], gd, sem)
                else:
                    d1 = pltpu.async_copy(ex_out.at[pl.ds(eb, K)], exb, sem)
                    d2 = None
                d3 = pltpu.async_copy(rb_hbm.at[rsrc_i.at[0]], gr, sem)
                if q == 1:
                    _scat(0, eb - K)
                else:
                    @pl.when(ch_i >= 1)
                    def _():
                        _scat(1, eb - K)
                d1.wait()
                if d2 is not None:
                    d2.wait()
                d3.wait()

                @plsc.parallel_loop(0, K, unroll=4)
                def _edge(k):
                    rrow = gr[k]
                    if p == 0:
                        e16 = gs[k] + _vperm(gd[k], shift8)
                        e16 = jnp.where(e16 >= 0, e16, 0.2 * e16)
                        ex = jnp.where(lo8, jnp.exp(e16), 0.0)
                        exb[k] = ex
                        for h in range(8):
                            w = _vperm(ex, lane0 + h)
                            msg[q * K + k, pl.ds(16 * h, 16)] = w * rrow
                    else:
                        ex = exb[k]
                        for h in range(8):
                            msg[q * K + k, pl.ds(16 * h, 16)] = ex[h] * rrow

                if p == 0:
                    pltpu.sync_copy(exb, den_sp.at[dst_i.at[q]], add=True)
                    pltpu.sync_copy(exb, ex_out.at[pl.ds(eb, K)])
            return 0

        lax.fori_loop(0, CH // 2, _iter, 0)
        _scat(1, ebase0 + (CH - 1) * K)
        plsc.subcore_barrier()

        pltpu.sync_copy(agg_sp.at[pl.ds(rows0, RPT)],
                        agg_out.at[c, p, pl.ds(rows0, RPT)])
        if p == 0:
            pltpu.sync_copy(den_sp.at[pl.ds(rows0, RPT)],
                            den_out.at[c, pl.ds(rows0, RPT)])
        plsc.subcore_barrier()


_sc2 = pl.kernel(
    _sc2_body,
    out_type=(
        jax.ShapeDtypeStruct((NC, 4, NP, 128), jnp.float32),
        jax.ShapeDtypeStruct((NC, NP, 16), jnp.float32),
        jax.ShapeDtypeStruct((EP, 16), jnp.float32),
    ),
    mesh=_mesh,
    scratch_types=(
        pltpu.VMEM_SHARED((NP, 128), jnp.float32),
        pltpu.VMEM_SHARED((NP, 16), jnp.float32),
        pltpu.VMEM((1, K), jnp.int32),
        pltpu.VMEM((2, K), jnp.int32),
        pltpu.VMEM((1, K), jnp.int32),
        pltpu.VMEM((K, 16), jnp.float32),
        pltpu.VMEM((K, 16), jnp.float32),
        pltpu.VMEM((K, 16), jnp.float32),
        pltpu.VMEM((K, 16), jnp.float32),
        pltpu.VMEM((2 * K, 128), jnp.float32),
        pltpu.VMEM((16, 128), jnp.float32),
        pltpu.VMEM((16, 16), jnp.float32),
        pltpu.SemaphoreType.DMA,
    ),
    compiler_params=pltpu.CompilerParams(use_tc_tiling_on_sc=False),
)


def _tc_fin_body(agg_ref, den_ref, w2_ref, b2_ref, out_ref):
    agg = agg_ref[0] + agg_ref[1]
    den = den_ref[0] + den_ref[1]
    inv = 0.125 / (den + 1e-16)
    acc = jnp.zeros((256, D_OUT), jnp.float32) + b2_ref[...]
    for h in range(HEADS):
        ah = jnp.concatenate(
            [agg[p, :, 16 * h:16 * h + 16] for p in range(4)], axis=1
        )
        ah = ah * inv[:, h:h + 1]
        acc = acc + jnp.dot(ah, w2_ref[:, 128 * h:128 * h + 128],
                            preferred_element_type=jnp.float32)
    out_ref[...] = acc


def _tc_fin(agg2, den2, W2, b2r):
    return pl.pallas_call(
        _tc_fin_body,
        grid=(NBLK,),
        in_specs=[
            pl.BlockSpec((NC, 4, 256, 128), lambda i: (0, 0, i, 0)),
            pl.BlockSpec((NC, 256, 16), lambda i: (0, i, 0)),
            pl.BlockSpec((64, HEADS * D_OUT), lambda i: (0, 0)),
            pl.BlockSpec((1, D_OUT), lambda i: (0, 0)),
        ],
        out_specs=pl.BlockSpec((256, D_OUT), lambda i: (i, 0)),
        out_shape=jax.ShapeDtypeStruct((NP, D_OUT), jnp.float32),
    )(agg2, den2, W2, b2r)


def kernel(x, edge_index, W1, att_src1, att_dst1, b1,
           W2, att_src2, att_dst2, b2):
    f32 = jnp.float32
    x = x.astype(f32)

    loop = jnp.arange(N, dtype=jnp.int32)
    src = jnp.concatenate([edge_index[0].astype(jnp.int32), loop])
    dst = jnp.concatenate([edge_index[1].astype(jnp.int32), loop])
    pad = EP - ET
    srcp = jnp.concatenate([src, jnp.zeros((pad,), jnp.int32)])
    dstp = jnp.concatenate([dst, jnp.full((pad,), NP - 1, jnp.int32)])

    hd = HEADS * DIM
    As = jnp.zeros((hd, HEADS), f32)
    Ad = jnp.zeros((hd, HEADS), f32)
    hh = jnp.arange(hd) // DIM
    As = As.at[jnp.arange(hd), hh].set(att_src1.reshape(-1).astype(f32))
    Ad = Ad.at[jnp.arange(hd), hh].set(att_dst1.reshape(-1).astype(f32))
    ho = HEADS * D_OUT
    hh2 = jnp.arange(ho) // D_OUT
    Ps = jnp.zeros((ho, HEADS), f32)
    Pd = jnp.zeros((ho, HEADS), f32)
    Ps = Ps.at[jnp.arange(ho), hh2].set(att_src2.reshape(-1).astype(f32))
    Pd = Pd.at[jnp.arange(ho), hh2].set(att_dst2.reshape(-1).astype(f32))
    Rep = jnp.zeros((16, 64), f32)
    Rep = Rep.at[jnp.arange(64) // DIM, jnp.arange(64)].set(1.0)

    xp = jnp.pad(x, ((0, NP - N), (0, 0)))
    h1p, asd1 = _tc_pre(xp, W1.astype(f32), As, Ad)

    (acc1,) = _sc1(srcp, dstp, asd1, h1p)

    rfull, asd2 = _tc_mid(acc1, b1.astype(f32).reshape(1, 64),
                          W2.astype(f32), Ps, Pd, Rep)
    rb = rfull.reshape(NP * 4, 16)

    agg2, den2, _ = _sc2(srcp, dstp, asd2, rb)

    outp = _tc_fin(agg2, den2, W2.astype(f32),
                   b2.astype(f32).reshape(1, D_OUT))
    return outp[:N]
